# baseline (speedup 1.0000x reference)
import functools

import jax
import jax.numpy as jnp
from jax import lax
from jax.experimental import pallas as pl
from jax.experimental.pallas import tpu as pltpu
from jax.experimental.pallas import tpu_sc as plsc

CAP = 100
LANES = 128
SEG = 256
CHUNK = 16
NC, NS = 2, 16


def _router_body(x_ref, wg_ref, bg_ref, p_ref, e_ref, hist_ref):
    logits = jnp.dot(x_ref[...], wg_ref[...], preferred_element_type=jnp.float32)
    logits = logits + bg_ref[...]
    m = jnp.max(logits, axis=1, keepdims=True)
    p_ref[...] = 1.0 / jnp.sum(jnp.exp(logits - m), axis=1, keepdims=True)
    col = lax.broadcasted_iota(jnp.int32, logits.shape, 1)
    e = jnp.min(jnp.where(logits == m, col, LANES), axis=1, keepdims=True)
    e_ref[...] = e
    hist_ref[...] = jnp.sum((col == e).astype(jnp.int32), axis=0).reshape(1, 1, LANES)


def _router(xf, wgp, bgp):
    t, h = xf.shape
    nseg = t // SEG
    return pl.pallas_call(
        _router_body,
        grid=(nseg,),
        in_specs=[
            pl.BlockSpec((SEG, h), lambda i: (i, 0)),
            pl.BlockSpec((h, LANES), lambda i: (0, 0)),
            pl.BlockSpec((1, LANES), lambda i: (0, 0)),
        ],
        out_specs=[
            pl.BlockSpec((SEG, 1), lambda i: (i, 0)),
            pl.BlockSpec((SEG, 1), lambda i: (i, 0)),
            pl.BlockSpec((1, 1, LANES), lambda i: (i, 0, 0)),
        ],
        out_shape=[
            jax.ShapeDtypeStruct((t, 1), jnp.float32),
            jax.ShapeDtypeStruct((t, 1), jnp.int32),
            jax.ShapeDtypeStruct((nseg, 1, LANES), jnp.int32),
        ],
    )(xf, wgp, bgp)


def _gate_body(n_experts, e_hbm, p_hbm, hist_hbm, g_hbm, e_v, p_v, hist_v, g_v):
    wid = lax.axis_index("s") * NC + lax.axis_index("c")
    base = wid * SEG
    pltpu.sync_copy(e_hbm.at[pl.ds(base, SEG)], e_v)
    pltpu.sync_copy(p_hbm.at[pl.ds(base, SEG)], p_v)
    pltpu.sync_copy(hist_hbm, hist_v)

    def off_body(w, acc):
        return acc + hist_v[pl.ds(w * LANES, CHUNK)]

    offs_vec = lax.fori_loop(0, wid, off_body, jnp.zeros((CHUNK,), jnp.int32))
    offs = tuple(offs_vec[ex] for ex in range(n_experts))

    def chunk_body(c, counts):
        e16 = e_v[pl.ds(c * CHUNK, CHUNK)]
        p16 = p_v[pl.ds(c * CHUNK, CHUNK)]
        g16 = jnp.zeros((CHUNK,), jnp.float32)
        new_counts = []
        for ex in range(n_experts):
            msk = e16 == ex
            cs = plsc.cumsum(msk.astype(jnp.int32))
            pos = counts[ex] + cs - 1
            keepm = jnp.logical_and(msk, pos < CAP)
            g16 = jnp.where(keepm, p16, g16)
            new_counts.append(counts[ex] + jnp.sum(msk.astype(jnp.int32)))
        g_v[pl.ds(c * CHUNK, CHUNK)] = g16
        return tuple(new_counts)

    lax.fori_loop(0, SEG // CHUNK, chunk_body, offs)
    pltpu.sync_copy(g_v, g_hbm.at[pl.ds(base, SEG)])


def _gate(e_flat, p_flat, hist_flat, n_experts):
    t = e_flat.shape[0]
    mesh = plsc.VectorSubcoreMesh(core_axis_name="c", subcore_axis_name="s")
    return pl.kernel(
        functools.partial(_gate_body, n_experts),
        out_type=jax.ShapeDtypeStruct((t,), jnp.float32),
        mesh=mesh,
        scratch_types=[
            pltpu.VMEM((SEG,), jnp.int32),
            pltpu.VMEM((SEG,), jnp.float32),
            pltpu.VMEM((hist_flat.shape[0],), jnp.int32),
            pltpu.VMEM((SEG,), jnp.float32),
        ],
        compiler_params=pltpu.CompilerParams(needs_layout_passes=False),
    )(e_flat, p_flat, hist_flat)


def _combine_body(x_ref, we_ref, be_ref, g_ref, o_ref):
    xb = x_ref[...].astype(jnp.bfloat16)
    acc = jnp.dot(xb, we_ref[...], preferred_element_type=jnp.float32)
    o_ref[...] = (acc + be_ref[...]) * g_ref[...]


def _combine(xf, we, be2, g2):
    t, h = xf.shape
    rb = 512
    return pl.pallas_call(
        _combine_body,
        grid=(t // rb,),
        in_specs=[
            pl.BlockSpec((rb, h), lambda i: (i, 0)),
            pl.BlockSpec((h, h), lambda i: (0, 0)),
            pl.BlockSpec((1, h), lambda i: (0, 0)),
            pl.BlockSpec((rb, 1), lambda i: (i, 0)),
        ],
        out_specs=pl.BlockSpec((rb, h), lambda i: (i, 0)),
        out_shape=jax.ShapeDtypeStruct((t, h), jnp.float32),
    )(xf, we, be2, g2)


def kernel(x, Wg, bg, We, be):
    b, s, h = x.shape
    n_experts = Wg.shape[1]
    t = b * s
    xf = x.reshape(t, h)
    wgp = jnp.zeros((h, LANES), jnp.float32).at[:, :n_experts].set(Wg)
    bgp = jnp.full((1, LANES), -1e30, jnp.float32).at[0, :n_experts].set(bg)
    p, e, hist = _router(xf, wgp, bgp)
    g = _gate(e.reshape(t), p.reshape(t), hist.reshape(-1), n_experts)
    out = _combine(xf, We.astype(jnp.bfloat16), be.reshape(1, h), g.reshape(t, 1))
    return out.reshape(b, s, h)


# speedup vs baseline: 1.1585x; 1.1585x over previous
import functools

import jax
import jax.numpy as jnp
from jax import lax
from jax.experimental import pallas as pl
from jax.experimental.pallas import tpu as pltpu
from jax.experimental.pallas import tpu_sc as plsc

CAP = 100
LANES = 128
SEG = 256
CHUNK = 16
NC, NS = 2, 16


def _fused_body(x_ref, wg_ref, bg_ref, we_ref, be_ref, p_ref, e_ref, hist_ref, y_ref):
    xb = x_ref[...]
    logits = jnp.dot(xb, wg_ref[...], preferred_element_type=jnp.float32)
    logits = logits + bg_ref[...]
    m = jnp.max(logits, axis=1, keepdims=True)
    p_ref[...] = 1.0 / jnp.sum(jnp.exp(logits - m), axis=1, keepdims=True)
    col = lax.broadcasted_iota(jnp.int32, logits.shape, 1)
    e = jnp.min(jnp.where(logits == m, col, LANES), axis=1, keepdims=True)
    e_ref[...] = e
    rows = xb.shape[0]
    nsub = rows // SEG
    seg_of = lax.broadcasted_iota(jnp.int32, (nsub, rows), 1) // SEG
    sel = (seg_of == lax.broadcasted_iota(jnp.int32, (nsub, rows), 0)).astype(jnp.float32)
    onehot = (col == e).astype(jnp.float32)
    hist = jnp.dot(sel, onehot, preferred_element_type=jnp.float32)
    hist_ref[...] = hist.astype(jnp.int32).reshape(nsub, 1, LANES)
    y = jnp.dot(xb.astype(jnp.bfloat16), we_ref[...],
                preferred_element_type=jnp.float32) + be_ref[...]
    y_ref[...] = y.astype(jnp.bfloat16)


def _fused(xf, wgp, bgp, web, be2):
    t, h = xf.shape
    rb = 1024
    nsub = rb // SEG
    return pl.pallas_call(
        _fused_body,
        grid=(t // rb,),
        in_specs=[
            pl.BlockSpec((rb, h), lambda i: (i, 0)),
            pl.BlockSpec((h, LANES), lambda i: (0, 0)),
            pl.BlockSpec((1, LANES), lambda i: (0, 0)),
            pl.BlockSpec((h, h), lambda i: (0, 0)),
            pl.BlockSpec((1, h), lambda i: (0, 0)),
        ],
        out_specs=[
            pl.BlockSpec((rb, 1), lambda i: (i, 0)),
            pl.BlockSpec((rb, 1), lambda i: (i, 0)),
            pl.BlockSpec((nsub, 1, LANES), lambda i: (i, 0, 0)),
            pl.BlockSpec((rb, h), lambda i: (i, 0)),
        ],
        out_shape=[
            jax.ShapeDtypeStruct((t, 1), jnp.float32),
            jax.ShapeDtypeStruct((t, 1), jnp.int32),
            jax.ShapeDtypeStruct((t // SEG, 1, LANES), jnp.int32),
            jax.ShapeDtypeStruct((t, h), jnp.bfloat16),
        ],
    )(xf, wgp, bgp, web, be2)


def _scale_body(y_ref, g_ref, o_ref):
    o_ref[...] = y_ref[...].astype(jnp.float32) * g_ref[...]


def _scale(y, g2):
    t, h = y.shape
    rb = 1024
    return pl.pallas_call(
        _scale_body,
        grid=(t // rb,),
        in_specs=[
            pl.BlockSpec((rb, h), lambda i: (i, 0)),
            pl.BlockSpec((rb, 1), lambda i: (i, 0)),
        ],
        out_specs=pl.BlockSpec((rb, h), lambda i: (i, 0)),
        out_shape=jax.ShapeDtypeStruct((t, h), jnp.float32),
    )(y, g2)


def _router_body(x_ref, wg_ref, bg_ref, p_ref, e_ref, hist_ref):
    logits = jnp.dot(x_ref[...], wg_ref[...], preferred_element_type=jnp.float32)
    logits = logits + bg_ref[...]
    m = jnp.max(logits, axis=1, keepdims=True)
    p_ref[...] = 1.0 / jnp.sum(jnp.exp(logits - m), axis=1, keepdims=True)
    col = lax.broadcasted_iota(jnp.int32, logits.shape, 1)
    e = jnp.min(jnp.where(logits == m, col, LANES), axis=1, keepdims=True)
    e_ref[...] = e
    hist_ref[...] = jnp.sum((col == e).astype(jnp.int32), axis=0).reshape(1, 1, LANES)


def _router(xf, wgp, bgp):
    t, h = xf.shape
    nseg = t // SEG
    return pl.pallas_call(
        _router_body,
        grid=(nseg,),
        in_specs=[
            pl.BlockSpec((SEG, h), lambda i: (i, 0)),
            pl.BlockSpec((h, LANES), lambda i: (0, 0)),
            pl.BlockSpec((1, LANES), lambda i: (0, 0)),
        ],
        out_specs=[
            pl.BlockSpec((SEG, 1), lambda i: (i, 0)),
            pl.BlockSpec((SEG, 1), lambda i: (i, 0)),
            pl.BlockSpec((1, 1, LANES), lambda i: (i, 0, 0)),
        ],
        out_shape=[
            jax.ShapeDtypeStruct((t, 1), jnp.float32),
            jax.ShapeDtypeStruct((t, 1), jnp.int32),
            jax.ShapeDtypeStruct((nseg, 1, LANES), jnp.int32),
        ],
    )(xf, wgp, bgp)


def _gate_body(n_experts, e_hbm, p_hbm, hist_hbm, g_hbm, e_v, p_v, hist_v, g_v):
    wid = lax.axis_index("s") * NC + lax.axis_index("c")
    base = wid * SEG
    pltpu.sync_copy(e_hbm.at[pl.ds(base, SEG)], e_v)
    pltpu.sync_copy(p_hbm.at[pl.ds(base, SEG)], p_v)
    pltpu.sync_copy(hist_hbm, hist_v)

    def off_body(w, acc):
        return acc + hist_v[pl.ds(w * LANES, CHUNK)]

    offs_vec = lax.fori_loop(0, wid, off_body, jnp.zeros((CHUNK,), jnp.int32))
    offs = tuple(offs_vec[ex] for ex in range(n_experts))

    def chunk_body(c, counts):
        e16 = e_v[pl.ds(c * CHUNK, CHUNK)]
        p16 = p_v[pl.ds(c * CHUNK, CHUNK)]
        g16 = jnp.zeros((CHUNK,), jnp.float32)
        new_counts = []
        for ex in range(n_experts):
            msk = e16 == ex
            cs = plsc.cumsum(msk.astype(jnp.int32))
            pos = counts[ex] + cs - 1
            keepm = jnp.logical_and(msk, pos < CAP)
            g16 = jnp.where(keepm, p16, g16)
            new_counts.append(counts[ex] + jnp.sum(msk.astype(jnp.int32)))
        g_v[pl.ds(c * CHUNK, CHUNK)] = g16
        return tuple(new_counts)

    lax.fori_loop(0, SEG // CHUNK, chunk_body, offs)
    pltpu.sync_copy(g_v, g_hbm.at[pl.ds(base, SEG)])


def _gate(e_flat, p_flat, hist_flat, n_experts):
    t = e_flat.shape[0]
    mesh = plsc.VectorSubcoreMesh(core_axis_name="c", subcore_axis_name="s")
    return pl.kernel(
        functools.partial(_gate_body, n_experts),
        out_type=jax.ShapeDtypeStruct((t,), jnp.float32),
        mesh=mesh,
        scratch_types=[
            pltpu.VMEM((SEG,), jnp.int32),
            pltpu.VMEM((SEG,), jnp.float32),
            pltpu.VMEM((hist_flat.shape[0],), jnp.int32),
            pltpu.VMEM((SEG,), jnp.float32),
        ],
        compiler_params=pltpu.CompilerParams(needs_layout_passes=False),
    )(e_flat, p_flat, hist_flat)


def _combine_body(x_ref, we_ref, be_ref, g_ref, o_ref):
    xb = x_ref[...].astype(jnp.bfloat16)
    acc = jnp.dot(xb, we_ref[...], preferred_element_type=jnp.float32)
    o_ref[...] = (acc + be_ref[...]) * g_ref[...]


def _combine(xf, we, be2, g2):
    t, h = xf.shape
    rb = 512
    return pl.pallas_call(
        _combine_body,
        grid=(t // rb,),
        in_specs=[
            pl.BlockSpec((rb, h), lambda i: (i, 0)),
            pl.BlockSpec((h, h), lambda i: (0, 0)),
            pl.BlockSpec((1, h), lambda i: (0, 0)),
            pl.BlockSpec((rb, 1), lambda i: (i, 0)),
        ],
        out_specs=pl.BlockSpec((rb, h), lambda i: (i, 0)),
        out_shape=jax.ShapeDtypeStruct((t, h), jnp.float32),
    )(xf, we, be2, g2)


def kernel(x, Wg, bg, We, be):
    b, s, h = x.shape
    n_experts = Wg.shape[1]
    t = b * s
    xf = x.reshape(t, h)
    wgp = jnp.zeros((h, LANES), jnp.float32).at[:, :n_experts].set(Wg)
    bgp = jnp.full((1, LANES), -1e30, jnp.float32).at[0, :n_experts].set(bg)
    p, e, hist, y = _fused(xf, wgp, bgp, We.astype(jnp.bfloat16), be.reshape(1, h))
    g = _gate(e.reshape(t), p.reshape(t), hist.reshape(-1), n_experts)
    out = _scale(y, g.reshape(t, 1))
    return out.reshape(b, s, h)


# speedup vs baseline: 1.2281x; 1.0601x over previous
import functools

import jax
import jax.numpy as jnp
from jax import lax
from jax.experimental import pallas as pl
from jax.experimental.pallas import tpu as pltpu
from jax.experimental.pallas import tpu_sc as plsc

CAP = 100
LANES = 128
SEG = 256
CHUNK = 16
NC, NS = 2, 16


def _router_body(x_ref, wg_ref, bg_ref, p_ref, e_ref, hist_ref, xb_ref):
    xb = x_ref[...]
    logits = jnp.dot(xb, wg_ref[...], preferred_element_type=jnp.float32)
    logits = logits + bg_ref[...]
    m = jnp.max(logits, axis=1, keepdims=True)
    p_ref[...] = 1.0 / jnp.sum(jnp.exp(logits - m), axis=1, keepdims=True)
    col = lax.broadcasted_iota(jnp.int32, logits.shape, 1)
    e = jnp.min(jnp.where(logits == m, col, LANES), axis=1, keepdims=True)
    e_ref[...] = e
    rows = xb.shape[0]
    nsub = rows // SEG
    seg_of = lax.broadcasted_iota(jnp.int32, (nsub, rows), 1) // SEG
    sel = (seg_of == lax.broadcasted_iota(jnp.int32, (nsub, rows), 0)).astype(jnp.float32)
    onehot = (col == e).astype(jnp.float32)
    hist = jnp.dot(sel, onehot, preferred_element_type=jnp.float32)
    hist_ref[...] = hist.astype(jnp.int32).reshape(nsub, 1, LANES)
    xb_ref[...] = xb.astype(jnp.bfloat16)


def _router(xf, wgp, bgp):
    t, h = xf.shape
    rb = 1024
    nsub = rb // SEG
    return pl.pallas_call(
        _router_body,
        grid=(t // rb,),
        in_specs=[
            pl.BlockSpec((rb, h), lambda i: (i, 0)),
            pl.BlockSpec((h, LANES), lambda i: (0, 0)),
            pl.BlockSpec((1, LANES), lambda i: (0, 0)),
        ],
        out_specs=[
            pl.BlockSpec((rb, 1), lambda i: (i, 0)),
            pl.BlockSpec((rb, 1), lambda i: (i, 0)),
            pl.BlockSpec((nsub, 1, LANES), lambda i: (i, 0, 0)),
            pl.BlockSpec((rb, h), lambda i: (i, 0)),
        ],
        out_shape=[
            jax.ShapeDtypeStruct((t, 1), jnp.float32),
            jax.ShapeDtypeStruct((t, 1), jnp.int32),
            jax.ShapeDtypeStruct((t // SEG, 1, LANES), jnp.int32),
            jax.ShapeDtypeStruct((t, h), jnp.bfloat16),
        ],
    )(xf, wgp, bgp)


def _gate_body(n_experts, e_hbm, p_hbm, hist_hbm, g_hbm, e_v, p_v, hist_v, g_v):
    wid = lax.axis_index("s") * NC + lax.axis_index("c")
    base = wid * SEG
    pltpu.sync_copy(e_hbm.at[pl.ds(base, SEG)], e_v)
    pltpu.sync_copy(p_hbm.at[pl.ds(base, SEG)], p_v)
    pltpu.sync_copy(hist_hbm, hist_v)

    def off_body(w, acc):
        return acc + hist_v[pl.ds(w * LANES, CHUNK)]

    offs_vec = lax.fori_loop(0, wid, off_body, jnp.zeros((CHUNK,), jnp.int32))
    offs = tuple(offs_vec[ex] for ex in range(n_experts))

    def chunk_body(c, counts):
        e16 = e_v[pl.ds(c * CHUNK, CHUNK)]
        p16 = p_v[pl.ds(c * CHUNK, CHUNK)]
        g16 = jnp.zeros((CHUNK,), jnp.float32)
        new_counts = []
        for ex in range(n_experts):
            msk = e16 == ex
            cs = plsc.cumsum(msk.astype(jnp.int32))
            pos = counts[ex] + cs - 1
            keepm = jnp.logical_and(msk, pos < CAP)
            g16 = jnp.where(keepm, p16, g16)
            new_counts.append(counts[ex] + jnp.sum(msk.astype(jnp.int32)))
        g_v[pl.ds(c * CHUNK, CHUNK)] = g16
        return tuple(new_counts)

    lax.fori_loop(0, SEG // CHUNK, chunk_body, offs)
    pltpu.sync_copy(g_v, g_hbm.at[pl.ds(base, SEG)])


def _gate(e_flat, p_flat, hist_flat, n_experts):
    t = e_flat.shape[0]
    mesh = plsc.VectorSubcoreMesh(core_axis_name="c", subcore_axis_name="s")
    return pl.kernel(
        functools.partial(_gate_body, n_experts),
        out_type=jax.ShapeDtypeStruct((t,), jnp.float32),
        mesh=mesh,
        scratch_types=[
            pltpu.VMEM((SEG,), jnp.int32),
            pltpu.VMEM((SEG,), jnp.float32),
            pltpu.VMEM((hist_flat.shape[0],), jnp.int32),
            pltpu.VMEM((SEG,), jnp.float32),
        ],
        compiler_params=pltpu.CompilerParams(needs_layout_passes=False),
    )(e_flat, p_flat, hist_flat)


def _combine_body(xb_ref, we_ref, be_ref, g_ref, o_ref):
    g = g_ref[...]
    anyk = jnp.max(g, axis=(0, 1))

    @pl.when(anyk > 0.0)
    def _():
        acc = jnp.dot(xb_ref[...], we_ref[...], preferred_element_type=jnp.float32)
        o_ref[...] = (acc + be_ref[...]) * g

    @pl.when(anyk <= 0.0)
    def _():
        o_ref[...] = jnp.zeros_like(o_ref)


def _combine(xb16, web, be2, g2):
    t, h = xb16.shape
    rb = 512
    return pl.pallas_call(
        _combine_body,
        grid=(t // rb,),
        in_specs=[
            pl.BlockSpec((rb, h), lambda i: (i, 0)),
            pl.BlockSpec((h, h), lambda i: (0, 0)),
            pl.BlockSpec((1, h), lambda i: (0, 0)),
            pl.BlockSpec((rb, 1), lambda i: (i, 0)),
        ],
        out_specs=pl.BlockSpec((rb, h), lambda i: (i, 0)),
        out_shape=jax.ShapeDtypeStruct((t, h), jnp.float32),
    )(xb16, web, be2, g2)


def kernel(x, Wg, bg, We, be):
    b, s, h = x.shape
    n_experts = Wg.shape[1]
    t = b * s
    xf = x.reshape(t, h)
    wgp = jnp.zeros((h, LANES), jnp.float32).at[:, :n_experts].set(Wg)
    bgp = jnp.full((1, LANES), -1e30, jnp.float32).at[0, :n_experts].set(bg)
    p, e, hist, xb16 = _router(xf, wgp, bgp)
    g = _gate(e.reshape(t), p.reshape(t), hist.reshape(-1), n_experts)
    out = _combine(xb16, We.astype(jnp.bfloat16), be.reshape(1, h), g.reshape(t, 1))
    return out.reshape(b, s, h)
